# baseline (speedup 1.0000x reference)
import functools

import jax
import jax.numpy as jnp
from jax import lax
from jax.experimental import pallas as pl
from jax.experimental.pallas import tpu as pltpu
from jax.experimental.pallas import tpu_sc as plsc

N = 10000
E = 320000
D = 128
DH = D // 2
DQ = D // 4
NC = 2
NS = 16
NW = NC * NS
CW = 128
NCHA = 160
NCHD = 80
EPAD = NS * NCHA * CW
NPAD = 10240
RPT = NPAD // NS
ROWBLK = 1024


_NBUF = 6
SPLIT = NPAD // NC
ACC_R = SPLIT + CW
RPS = SPLIT // NS
NFLAT = (NCHA + 1) * CW


def _adj_body(y4_hbm, row_hbm, col_hbm, out_hbm,
              row_v, col_v, colx, b0, b1, b2, b3, b4, b5,
              g0, g1, g2, g3, g4, g5, s0, s1, s2, s3, s4, s5,
              ytab, accum):
    c = lax.axis_index("c")
    s = lax.axis_index("s")
    bufs = (b0, b1, b2, b3, b4, b5)
    gsem = (g0, g1, g2, g3, g4, g5)
    ssem = (s0, s1, s2, s3, s4, s5)

    pltpu.sync_copy(row_hbm.at[s], row_v)
    pltpu.sync_copy(col_hbm.at[s], col_v)

    for q in range(2):
        qi = c * 2 + q

        for k in range(RPT // CW):
            base = s * RPT + k * CW
            pltpu.sync_copy(y4_hbm.at[pl.ds(base, CW), qi], b0)
            pltpu.sync_copy(b0, ytab.at[pl.ds(base, CW)])
        plsc.subcore_barrier()

        for h in range(NC):
            lo = h * SPLIT

            def remap(j, carry):
                for k in range(CW // 16):
                    v = col_v[j, pl.ds(k * 16, 16)]
                    m = jnp.logical_and(v >= lo, v < lo + SPLIT)
                    dump = SPLIT + (k % 8) * 16 + lax.iota(jnp.int32, 16)
                    colx[j, pl.ds(k * 16, 16)] = jnp.where(m, v - lo, dump)
                return carry

            lax.fori_loop(0, NCHA, remap, 0)

            for k in range(RPS // CW + 1):
                n = CW if k < RPS // CW else RPS - (RPS // CW) * CW
                if n == 0:
                    continue
                base = s * RPS + k * CW
                pltpu.sync_copy(ytab.at[pl.ds(lo + base, n)],
                                b0.at[pl.ds(0, n)])
                pltpu.sync_copy(b0.at[pl.ds(0, n)], accum.at[pl.ds(base, n)])
            plsc.subcore_barrier()

            def ring(i, carry):
                for b in range(_NBUF):
                    j = i * _NBUF + b

                    @pl.when(jnp.logical_and(j >= _NBUF, j < NCHA + _NBUF))
                    def _():
                        pltpu.make_async_copy(
                            bufs[b], accum.at[colx.at[j - _NBUF]],
                            ssem[b]).wait()

                    @pl.when(j < NCHA)
                    def _():
                        pltpu.async_copy(
                            ytab.at[row_v.at[j]], bufs[b], gsem[b])

                    bp = (b + _NBUF - 2) % _NBUF

                    @pl.when(jnp.logical_and(j >= 2, j < NCHA + 2))
                    def _():
                        pltpu.make_async_copy(
                            ytab.at[row_v.at[j - 2]], bufs[bp],
                            gsem[bp]).wait()
                        pltpu.async_copy(
                            bufs[bp], accum.at[colx.at[j - 2]], ssem[bp],
                            add=True)

                return carry

            lax.fori_loop(0, (NCHA + 2 * _NBUF) // _NBUF, ring, 0)
            plsc.subcore_barrier()

            for k in range(RPS // CW + 1):
                n = CW if k < RPS // CW else RPS - (RPS // CW) * CW
                if n == 0:
                    continue
                base = s * RPS + k * CW
                pltpu.sync_copy(accum.at[pl.ds(base, n)], b0.at[pl.ds(0, n)])
                pltpu.sync_copy(b0.at[pl.ds(0, n)],
                                out_hbm.at[pl.ds(lo + base, n), qi])


_adj_pass = functools.partial(
    pl.kernel,
    out_type=jax.ShapeDtypeStruct((NPAD, 4, DQ), jnp.float32),
    mesh=plsc.VectorSubcoreMesh(core_axis_name="c", subcore_axis_name="s"),
    scratch_types=[
        pltpu.VMEM((NCHA, CW), jnp.int32),
        pltpu.VMEM((NCHA, CW), jnp.int32),
        pltpu.VMEM((NCHA, CW), jnp.int32),
        pltpu.VMEM((CW, DQ), jnp.float32),
        pltpu.VMEM((CW, DQ), jnp.float32),
        pltpu.VMEM((CW, DQ), jnp.float32),
        pltpu.VMEM((CW, DQ), jnp.float32),
        pltpu.VMEM((CW, DQ), jnp.float32),
        pltpu.VMEM((CW, DQ), jnp.float32),
        pltpu.SemaphoreType.DMA,
        pltpu.SemaphoreType.DMA,
        pltpu.SemaphoreType.DMA,
        pltpu.SemaphoreType.DMA,
        pltpu.SemaphoreType.DMA,
        pltpu.SemaphoreType.DMA,
        pltpu.SemaphoreType.DMA,
        pltpu.SemaphoreType.DMA,
        pltpu.SemaphoreType.DMA,
        pltpu.SemaphoreType.DMA,
        pltpu.SemaphoreType.DMA,
        pltpu.SemaphoreType.DMA,
        pltpu.VMEM_SHARED((NPAD, DQ), jnp.float32),
        pltpu.VMEM_SHARED((ACC_R, DQ), jnp.float32),
    ],
    compiler_params=pltpu.CompilerParams(use_tc_tiling_on_sc=False),
)(_adj_body)


def _deg_body(col_hbm, out_hbm, col_v, ones_v, zbuf, accum):
    c = lax.axis_index("c")
    s = lax.axis_index("s")
    wid = s * NC + c

    pltpu.sync_copy(col_hbm.at[wid], col_v)
    for j in range(CW // 16):
        ones_v[pl.ds(j * 16, 16)] = jnp.ones((16,), jnp.float32)
    for j in range(RPT // 16):
        zbuf[pl.ds(j * 16, 16)] = jnp.zeros((16,), jnp.float32)
    pltpu.sync_copy(zbuf, accum.at[pl.ds(s * RPT, RPT)])
    plsc.subcore_barrier()

    def step(j, carry):
        pltpu.sync_copy(ones_v, accum.at[col_v.at[j]], add=True)
        return carry

    lax.fori_loop(0, NCHD, step, 0)

    plsc.subcore_barrier()
    pltpu.sync_copy(accum.at[pl.ds(s * RPT, RPT)], zbuf)
    pltpu.sync_copy(zbuf, out_hbm.at[c].at[pl.ds(s * RPT, RPT)])


_deg_pass = functools.partial(
    pl.kernel,
    out_type=jax.ShapeDtypeStruct((NC, NPAD), jnp.float32),
    mesh=plsc.VectorSubcoreMesh(core_axis_name="c", subcore_axis_name="s"),
    scratch_types=[
        pltpu.VMEM((NCHD, CW), jnp.int32),
        pltpu.VMEM((CW,), jnp.float32),
        pltpu.VMEM((RPT,), jnp.float32),
        pltpu.VMEM_SHARED((NPAD,), jnp.float32),
    ],
)(_deg_body)


_NB = NPAD // ROWBLK

_vec_spec = pl.BlockSpec((ROWBLK,), lambda i: (i,))
_mat_spec = pl.BlockSpec((ROWBLK, D), lambda i: (i, 0))
_w_spec = pl.BlockSpec((D, D), lambda i: (0, 0))
_b_spec = pl.BlockSpec((D,), lambda i: (0,))


def _scale_body(d0_ref, d1_ref, x_ref, s_ref, a_ref):
    d = d0_ref[...] + d1_ref[...] + 1.0
    sv = lax.rsqrt(d)
    s_ref[...] = sv
    a_ref[...] = x_ref[...] * sv[:, None]


_k_scale = pl.pallas_call(
    _scale_body,
    grid=(_NB,),
    in_specs=[_vec_spec, _vec_spec, _mat_spec],
    out_specs=[_vec_spec, _mat_spec],
    out_shape=[
        jax.ShapeDtypeStruct((NPAD,), jnp.float32),
        jax.ShapeDtypeStruct((NPAD, D), jnp.float32),
    ],
)


def _combine_body(s_ref, z_ref, o_ref):
    sv = s_ref[...]
    o_ref[...] = z_ref[...] * (sv * sv)[:, None]


_k_combine = pl.pallas_call(
    _combine_body,
    grid=(_NB,),
    in_specs=[_vec_spec, _mat_spec],
    out_specs=_mat_spec,
    out_shape=jax.ShapeDtypeStruct((NPAD, D), jnp.float32),
)


def _mm_relu_body(s_ref, z_ref, w_ref, b_ref, o_ref):
    sv = s_ref[...]
    t = z_ref[...] * sv[:, None]
    m = jnp.dot(t, w_ref[...], preferred_element_type=jnp.float32)
    m = m + b_ref[...][None, :]
    o_ref[...] = jnp.maximum(m, 0.0) * sv[:, None]


_k_mm_relu = pl.pallas_call(
    _mm_relu_body,
    grid=(_NB,),
    in_specs=[_vec_spec, _mat_spec, _w_spec, _b_spec],
    out_specs=_mat_spec,
    out_shape=jax.ShapeDtypeStruct((NPAD, D), jnp.float32),
)


def _mm_body(s_ref, z_ref, w_ref, b_ref, o_ref):
    sv = s_ref[...]
    t = z_ref[...] * sv[:, None]
    m = jnp.dot(t, w_ref[...], preferred_element_type=jnp.float32)
    o_ref[...] = m + b_ref[...][None, :]


_k_mm = pl.pallas_call(
    _mm_body,
    grid=(_NB,),
    in_specs=[_vec_spec, _mat_spec, _w_spec, _b_spec],
    out_specs=_mat_spec,
    out_shape=jax.ShapeDtypeStruct((NPAD, D), jnp.float32),
)


def _adj(y, row_a, col_a):
    y4 = y.reshape(NPAD, 4, DQ)
    z4 = _adj_pass(y4, row_a, col_a)
    return z4.reshape(NPAD, D)


def kernel(x, edge_index, W1, b1, W2, b2):
    ei = edge_index.astype(jnp.int32)
    pad = EPAD - E
    row = jnp.concatenate([ei[0], jnp.zeros((pad,), jnp.int32)])
    col = jnp.concatenate([ei[1], jnp.full((pad,), N, jnp.int32)])
    row_a = row.reshape(NS, NCHA, CW)
    col_a = col.reshape(NS, NCHA, CW)
    col_d = col.reshape(NW, NCHD, CW)

    xp = jnp.concatenate([x, jnp.zeros((NPAD - N, D), jnp.float32)], axis=0)

    degp = _deg_pass(col_d)
    sv, a = _k_scale(degp[0], degp[1], xp)

    b = _adj(a, row_a, col_a)
    cc = _k_combine(sv, b)
    d = _adj(cc, row_a, col_a)
    e = _k_mm_relu(sv, d, W1, b1)
    f = _adj(e, row_a, col_a)
    g = _k_combine(sv, f)
    h = _adj(g, row_a, col_a)
    out = _k_mm(sv, h, W2, b2)

    return out[:N]


# speedup vs baseline: 1.0032x; 1.0032x over previous
import functools

import jax
import jax.numpy as jnp
from jax import lax
from jax.experimental import pallas as pl
from jax.experimental.pallas import tpu as pltpu
from jax.experimental.pallas import tpu_sc as plsc

N = 10000
E = 320000
D = 128
DH = D // 2
DQ = D // 4
NC = 2
NS = 16
NW = NC * NS
CW = 128
NCHA = 160
NCHD = 80
EPAD = NS * NCHA * CW
NPAD = 10240
RPT = NPAD // NS
ROWBLK = 1024


_NBUF = 4
SPLIT = NPAD // NC
ACC_R = SPLIT + CW
RPS = SPLIT // NS
NFLAT = (NCHA + 1) * CW


def _adj_body(y4_hbm, row_hbm, col_hbm, out_hbm,
              row_v, col_v, colx, b0, b1, b2, b3,
              g0, g1, g2, g3, s0, s1, s2, s3, ytab, accum):
    c = lax.axis_index("c")
    s = lax.axis_index("s")
    bufs = (b0, b1, b2, b3)
    gsem = (g0, g1, g2, g3)
    ssem = (s0, s1, s2, s3)

    pltpu.sync_copy(row_hbm.at[s], row_v)
    pltpu.sync_copy(col_hbm.at[s], col_v)

    for q in range(2):
        qi = c * 2 + q

        for k in range(RPT // CW):
            base = s * RPT + k * CW
            pltpu.sync_copy(y4_hbm.at[pl.ds(base, CW), qi], b0)
            pltpu.sync_copy(b0, ytab.at[pl.ds(base, CW)])
        plsc.subcore_barrier()

        for h in range(NC):
            lo = h * SPLIT

            def remap(j, carry):
                for k in range(CW // 16):
                    v = col_v[j, pl.ds(k * 16, 16)]
                    m = jnp.logical_and(v >= lo, v < lo + SPLIT)
                    dump = SPLIT + (k % 8) * 16 + lax.iota(jnp.int32, 16)
                    colx[j, pl.ds(k * 16, 16)] = jnp.where(m, v - lo, dump)
                return carry

            lax.fori_loop(0, NCHA, remap, 0)

            for k in range(RPS // CW + 1):
                n = CW if k < RPS // CW else RPS - (RPS // CW) * CW
                if n == 0:
                    continue
                base = s * RPS + k * CW
                pltpu.sync_copy(ytab.at[pl.ds(lo + base, n)],
                                b0.at[pl.ds(0, n)])
                pltpu.sync_copy(b0.at[pl.ds(0, n)], accum.at[pl.ds(base, n)])
            plsc.subcore_barrier()

            def ring(i, carry):
                for b in range(_NBUF):
                    j = i * _NBUF + b

                    @pl.when(jnp.logical_and(j >= _NBUF, j < NCHA + _NBUF))
                    def _():
                        pltpu.make_async_copy(
                            bufs[b], accum.at[colx.at[j - _NBUF]],
                            ssem[b]).wait()

                    @pl.when(j < NCHA)
                    def _():
                        pltpu.async_copy(
                            ytab.at[row_v.at[j]], bufs[b], gsem[b])

                    bp = (b + _NBUF - 2) % _NBUF

                    @pl.when(jnp.logical_and(j >= 2, j < NCHA + 2))
                    def _():
                        pltpu.make_async_copy(
                            ytab.at[row_v.at[j - 2]], bufs[bp],
                            gsem[bp]).wait()
                        pltpu.async_copy(
                            bufs[bp], accum.at[colx.at[j - 2]], ssem[bp],
                            add=True)

                return carry

            lax.fori_loop(0, (NCHA + 2 * _NBUF) // _NBUF, ring, 0)
            plsc.subcore_barrier()

            for k in range(RPS // CW + 1):
                n = CW if k < RPS // CW else RPS - (RPS // CW) * CW
                if n == 0:
                    continue
                base = s * RPS + k * CW
                pltpu.sync_copy(accum.at[pl.ds(base, n)], b0.at[pl.ds(0, n)])
                pltpu.sync_copy(b0.at[pl.ds(0, n)],
                                out_hbm.at[pl.ds(lo + base, n), qi])


_adj_pass = functools.partial(
    pl.kernel,
    out_type=jax.ShapeDtypeStruct((NPAD, 4, DQ), jnp.float32),
    mesh=plsc.VectorSubcoreMesh(core_axis_name="c", subcore_axis_name="s"),
    scratch_types=[
        pltpu.VMEM((NCHA, CW), jnp.int32),
        pltpu.VMEM((NCHA, CW), jnp.int32),
        pltpu.VMEM((NCHA, CW), jnp.int32),
        pltpu.VMEM((CW, DQ), jnp.float32),
        pltpu.VMEM((CW, DQ), jnp.float32),
        pltpu.VMEM((CW, DQ), jnp.float32),
        pltpu.VMEM((CW, DQ), jnp.float32),
        pltpu.SemaphoreType.DMA,
        pltpu.SemaphoreType.DMA,
        pltpu.SemaphoreType.DMA,
        pltpu.SemaphoreType.DMA,
        pltpu.SemaphoreType.DMA,
        pltpu.SemaphoreType.DMA,
        pltpu.SemaphoreType.DMA,
        pltpu.SemaphoreType.DMA,
        pltpu.VMEM_SHARED((NPAD, DQ), jnp.float32),
        pltpu.VMEM_SHARED((ACC_R, DQ), jnp.float32),
    ],
    compiler_params=pltpu.CompilerParams(use_tc_tiling_on_sc=False),
)(_adj_body)


def _deg_body(col_hbm, out_hbm, col_v, ones_v, zbuf, accum):
    c = lax.axis_index("c")
    s = lax.axis_index("s")
    wid = s * NC + c

    pltpu.sync_copy(col_hbm.at[wid], col_v)
    for j in range(CW // 16):
        ones_v[pl.ds(j * 16, 16)] = jnp.ones((16,), jnp.float32)
    for j in range(RPT // 16):
        zbuf[pl.ds(j * 16, 16)] = jnp.zeros((16,), jnp.float32)
    pltpu.sync_copy(zbuf, accum.at[pl.ds(s * RPT, RPT)])
    plsc.subcore_barrier()

    def step(j, carry):
        pltpu.sync_copy(ones_v, accum.at[col_v.at[j]], add=True)
        return carry

    lax.fori_loop(0, NCHD, step, 0)

    plsc.subcore_barrier()
    pltpu.sync_copy(accum.at[pl.ds(s * RPT, RPT)], zbuf)
    pltpu.sync_copy(zbuf, out_hbm.at[c].at[pl.ds(s * RPT, RPT)])


_deg_pass = functools.partial(
    pl.kernel,
    out_type=jax.ShapeDtypeStruct((NC, NPAD), jnp.float32),
    mesh=plsc.VectorSubcoreMesh(core_axis_name="c", subcore_axis_name="s"),
    scratch_types=[
        pltpu.VMEM((NCHD, CW), jnp.int32),
        pltpu.VMEM((CW,), jnp.float32),
        pltpu.VMEM((RPT,), jnp.float32),
        pltpu.VMEM_SHARED((NPAD,), jnp.float32),
    ],
)(_deg_body)


_NB = NPAD // ROWBLK

_vec_spec = pl.BlockSpec((ROWBLK,), lambda i: (i,))
_mat_spec = pl.BlockSpec((ROWBLK, D), lambda i: (i, 0))
_w_spec = pl.BlockSpec((D, D), lambda i: (0, 0))
_b_spec = pl.BlockSpec((D,), lambda i: (0,))


def _scale_body(d0_ref, d1_ref, x_ref, s_ref, a_ref):
    d = d0_ref[...] + d1_ref[...] + 1.0
    sv = lax.rsqrt(d)
    s_ref[...] = sv
    a_ref[...] = x_ref[...] * sv[:, None]


_k_scale = pl.pallas_call(
    _scale_body,
    grid=(_NB,),
    in_specs=[_vec_spec, _vec_spec, _mat_spec],
    out_specs=[_vec_spec, _mat_spec],
    out_shape=[
        jax.ShapeDtypeStruct((NPAD,), jnp.float32),
        jax.ShapeDtypeStruct((NPAD, D), jnp.float32),
    ],
)


def _combine_body(s_ref, z_ref, o_ref):
    sv = s_ref[...]
    o_ref[...] = z_ref[...] * (sv * sv)[:, None]


_k_combine = pl.pallas_call(
    _combine_body,
    grid=(_NB,),
    in_specs=[_vec_spec, _mat_spec],
    out_specs=_mat_spec,
    out_shape=jax.ShapeDtypeStruct((NPAD, D), jnp.float32),
)


def _mm_relu_body(s_ref, z_ref, w_ref, b_ref, o_ref):
    sv = s_ref[...]
    t = z_ref[...] * sv[:, None]
    m = jnp.dot(t, w_ref[...], preferred_element_type=jnp.float32)
    m = m + b_ref[...][None, :]
    o_ref[...] = jnp.maximum(m, 0.0) * sv[:, None]


_k_mm_relu = pl.pallas_call(
    _mm_relu_body,
    grid=(_NB,),
    in_specs=[_vec_spec, _mat_spec, _w_spec, _b_spec],
    out_specs=_mat_spec,
    out_shape=jax.ShapeDtypeStruct((NPAD, D), jnp.float32),
)


def _mm_body(s_ref, z_ref, w_ref, b_ref, o_ref):
    sv = s_ref[...]
    t = z_ref[...] * sv[:, None]
    m = jnp.dot(t, w_ref[...], preferred_element_type=jnp.float32)
    o_ref[...] = m + b_ref[...][None, :]


_k_mm = pl.pallas_call(
    _mm_body,
    grid=(_NB,),
    in_specs=[_vec_spec, _mat_spec, _w_spec, _b_spec],
    out_specs=_mat_spec,
    out_shape=jax.ShapeDtypeStruct((NPAD, D), jnp.float32),
)


def _adj(y, row_a, col_a):
    y4 = y.reshape(NPAD, 4, DQ)
    z4 = _adj_pass(y4, row_a, col_a)
    return z4.reshape(NPAD, D)


def kernel(x, edge_index, W1, b1, W2, b2):
    ei = edge_index.astype(jnp.int32)
    pad = EPAD - E
    row = jnp.concatenate([ei[0], jnp.zeros((pad,), jnp.int32)])
    col = jnp.concatenate([ei[1], jnp.full((pad,), N, jnp.int32)])
    row_a = row.reshape(NS, NCHA, CW)
    col_a = col.reshape(NS, NCHA, CW)
    col_d = col.reshape(NW, NCHD, CW)

    xp = jnp.concatenate([x, jnp.zeros((NPAD - N, D), jnp.float32)], axis=0)

    degp = _deg_pass(col_d)
    sv, a = _k_scale(degp[0], degp[1], xp)

    b = _adj(a, row_a, col_a)
    cc = _k_combine(sv, b)
    d = _adj(cc, row_a, col_a)
    e = _k_mm_relu(sv, d, W1, b1)
    f = _adj(e, row_a, col_a)
    g = _k_combine(sv, f)
    h = _adj(g, row_a, col_a)
    out = _k_mm(sv, h, W2, b2)

    return out[:N]
